# baseline (speedup 1.0000x reference)
import functools

import jax
import jax.numpy as jnp
from jax import lax
from jax.experimental import pallas as pl
from jax.experimental.pallas import tpu as pltpu
from jax.experimental.pallas import tpu_sc as plsc

VOCAB = 100000
D = 300
DP = 304
B = 4096
S = 50
NTOT = B * S
NW = 32
PER_W = NTOT // NW
CH = 128
NCHUNK = PER_W // CH


def _make_gather():
    mesh = plsc.VectorSubcoreMesh(core_axis_name="c", subcore_axis_name="s")

    @functools.partial(
        pl.kernel,
        mesh=mesh,
        out_type=jax.ShapeDtypeStruct((NTOT, DP), jnp.float32),
        scratch_types=[
            pltpu.VMEM((NCHUNK, CH), jnp.int32),
            pltpu.VMEM((2, CH, DP), jnp.float32),
            pltpu.SemaphoreType.DMA,
            pltpu.SemaphoreType.DMA,
        ],
        compiler_params=pltpu.CompilerParams(use_tc_tiling_on_sc=False),
    )
    def gather_kernel(idx_hbm, table_hbm, out_hbm, idx_v, rows_v, sem0, sem1):
        wid = lax.axis_index("s") * 2 + lax.axis_index("c")
        base = wid * PER_W
        pltpu.sync_copy(idx_hbm.at[pl.ds(wid * NCHUNK, NCHUNK)], idx_v)

        sems = (sem0, sem1)

        def start(c, buf):
            pltpu.async_copy(
                table_hbm.at[idx_v.at[c]],
                rows_v.at[buf],
                sems[buf],
            )

        def wait(c, buf):
            pltpu.make_async_copy(
                table_hbm.at[idx_v.at[c]],
                rows_v.at[buf],
                sems[buf],
            ).wait()

        start(0, 0)
        start(1, 1)

        def body(i, _):
            c0 = i * 2
            for b in range(2):
                c = c0 + b
                wait(c, b)
                pltpu.sync_copy(rows_v.at[b],
                                out_hbm.at[pl.ds(base + c * CH, CH)])

                @pl.when(c + 2 < NCHUNK)
                def _():
                    start(c + 2, b)
            return 0

        lax.fori_loop(0, NCHUNK // 2, body, 0)

    return gather_kernel


_gather = _make_gather()


def kernel(indices, table):
    idx_2d = indices.reshape(NW * NCHUNK, CH).astype(jnp.int32)
    table_p = jnp.pad(table, ((0, 0), (0, DP - D)))
    out_p = _gather(idx_2d, table_p)
    return out_p[:, :D].reshape(B, S, D)


# speedup vs baseline: 1.3561x; 1.3561x over previous
import functools

import jax
import jax.numpy as jnp
from jax import lax
from jax.experimental import pallas as pl
from jax.experimental.pallas import tpu as pltpu
from jax.experimental.pallas import tpu_sc as plsc

VOCAB = 100000
D = 300
DP = 384
B = 4096
S = 50
NTOT = B * S
NW = 32
PER_W = NTOT // NW
CH = 128
NCHUNK = PER_W // CH


def _make_gather():
    mesh = plsc.VectorSubcoreMesh(core_axis_name="c", subcore_axis_name="s")

    @functools.partial(
        pl.kernel,
        mesh=mesh,
        out_type=jax.ShapeDtypeStruct((NTOT, DP), jnp.float32),
        scratch_types=[
            pltpu.VMEM((PER_W,), jnp.int32),
            pltpu.VMEM((2, CH, DP), jnp.float32),
            pltpu.SemaphoreType.DMA,
            pltpu.SemaphoreType.DMA,
        ],
        compiler_params=pltpu.CompilerParams(use_tc_tiling_on_sc=True),
    )
    def gather_kernel(idx_hbm, table_hbm, out_hbm, idx_v, rows_v, sem0, sem1):
        wid = lax.axis_index("s") * 2 + lax.axis_index("c")
        base = wid * PER_W
        pltpu.sync_copy(idx_hbm.at[pl.ds(base, PER_W)], idx_v)

        sems = (sem0, sem1)

        def start(c, buf):
            pltpu.async_copy(
                table_hbm.at[idx_v.at[pl.ds(c * CH, CH)]],
                rows_v.at[buf],
                sems[buf],
            )

        def wait(c, buf):
            pltpu.make_async_copy(
                table_hbm.at[idx_v.at[pl.ds(c * CH, CH)]],
                rows_v.at[buf],
                sems[buf],
            ).wait()

        start(0, 0)
        start(1, 1)

        def body(i, _):
            c0 = i * 2
            for b in range(2):
                c = c0 + b
                wait(c, b)
                pltpu.sync_copy(rows_v.at[b],
                                out_hbm.at[pl.ds(base + c * CH, CH)])

                @pl.when(c + 2 < NCHUNK)
                def _():
                    start(c + 2, b)
            return 0

        lax.fori_loop(0, NCHUNK // 2, body, 0)

    return gather_kernel


_gather = _make_gather()


def kernel(indices, table):
    idx_flat = indices.reshape(-1).astype(jnp.int32)
    table_p = jnp.pad(table, ((0, 0), (0, DP - D)))
    out_p = _gather(idx_flat, table_p)
    return out_p[:, :D].reshape(B, S, D)


# speedup vs baseline: 1.5432x; 1.1379x over previous
import functools

import jax
import jax.numpy as jnp
from jax import lax
from jax.experimental import pallas as pl
from jax.experimental.pallas import tpu as pltpu
from jax.experimental.pallas import tpu_sc as plsc

VOCAB = 100000
D = 300
DP = 384
B = 4096
S = 50
NTOT = B * S
NW = 32
PER_W = NTOT // NW
CH = 128
NCHUNK = PER_W // CH


def _make_gather():
    mesh = plsc.VectorSubcoreMesh(core_axis_name="c", subcore_axis_name="s")

    @functools.partial(
        pl.kernel,
        mesh=mesh,
        out_type=jax.ShapeDtypeStruct((NTOT, DP), jnp.float32),
        scratch_types=[
            pltpu.VMEM((PER_W,), jnp.int32),
            pltpu.VMEM((2, CH, DP), jnp.float32),
            pltpu.SemaphoreType.DMA,
            pltpu.SemaphoreType.DMA,
        ],
        compiler_params=pltpu.CompilerParams(use_tc_tiling_on_sc=True),
    )
    def gather_kernel(idx_hbm, table_hbm, out_hbm, idx_v, rows_v, sem0, sem1):
        wid = lax.axis_index("s") * 2 + lax.axis_index("c")
        base = wid * PER_W
        pltpu.sync_copy(idx_hbm.at[pl.ds(base, PER_W)], idx_v)

        sems = (sem0, sem1)

        def start(c, buf):
            pltpu.async_copy(
                table_hbm.at[idx_v.at[pl.ds(c * CH, CH)]],
                rows_v.at[buf],
                sems[buf],
            )

        def wait(c, buf):
            pltpu.make_async_copy(
                table_hbm.at[idx_v.at[pl.ds(c * CH, CH)]],
                rows_v.at[buf],
                sems[buf],
            ).wait()

        start(0, 0)
        start(1, 1)

        def body(i, _):
            c0 = i * 2
            for b in range(2):
                c = c0 + b
                wait(c, b)
                pltpu.sync_copy(rows_v.at[b],
                                out_hbm.at[pl.ds(base + c * CH, CH)])

                @pl.when(c + 2 < NCHUNK)
                def _():
                    start(c + 2, b)
            return 0

        lax.fori_loop(0, NCHUNK // 2, body, 0)

    return gather_kernel


_gather = _make_gather()


_PAD_R = 2000
_STRIP_R = 2048


def _pad_body(x_ref, o_ref):
    o_ref[...] = jnp.pad(x_ref[...], ((0, 0), (0, DP - D)))


_pad_tc = pl.pallas_call(
    _pad_body,
    grid=(VOCAB // _PAD_R,),
    in_specs=[pl.BlockSpec((_PAD_R, D), lambda g: (g, 0))],
    out_specs=pl.BlockSpec((_PAD_R, DP), lambda g: (g, 0)),
    out_shape=jax.ShapeDtypeStruct((VOCAB, DP), jnp.float32),
)


def _strip_body(x_ref, o_ref):
    o_ref[...] = x_ref[:, :D]


_strip_tc = pl.pallas_call(
    _strip_body,
    grid=(NTOT // _STRIP_R,),
    in_specs=[pl.BlockSpec((_STRIP_R, DP), lambda g: (g, 0))],
    out_specs=pl.BlockSpec((_STRIP_R, D), lambda g: (g, 0)),
    out_shape=jax.ShapeDtypeStruct((NTOT, D), jnp.float32),
)


def kernel(indices, table):
    idx_flat = indices.reshape(-1).astype(jnp.int32)
    table_p = _pad_tc(table)
    out_p = _gather(idx_flat, table_p)
    return _strip_tc(out_p).reshape(B, S, D)


# speedup vs baseline: 3.5406x; 2.2944x over previous
import functools

import jax
import jax.numpy as jnp
from jax import lax
from jax.experimental import pallas as pl
from jax.experimental.pallas import tpu as pltpu
from jax.experimental.pallas import tpu_sc as plsc

VOCAB = 100000
D = 300
DP = 384
B = 4096
S = 50
NTOT = B * S
NW = 32
PER_W = NTOT // NW
CH = 128
NCHUNK = PER_W // CH


def _make_gather():
    mesh = plsc.VectorSubcoreMesh(core_axis_name="c", subcore_axis_name="s")

    @functools.partial(
        pl.kernel,
        mesh=mesh,
        out_type=jax.ShapeDtypeStruct((NTOT, DP), jnp.float32),
        scratch_types=[
            pltpu.VMEM((PER_W,), jnp.int32),
            pltpu.VMEM((2, CH, DP), jnp.float32),
            pltpu.SemaphoreType.DMA,
            pltpu.SemaphoreType.DMA,
        ],
        compiler_params=pltpu.CompilerParams(use_tc_tiling_on_sc=True),
    )
    def gather_kernel(idx_hbm, table_hbm, out_hbm, idx_v, rows_v, sem0, sem1):
        wid = lax.axis_index("s") * 2 + lax.axis_index("c")
        base = wid * PER_W
        pltpu.sync_copy(idx_hbm.at[pl.ds(base, PER_W)], idx_v)

        sems = (sem0, sem1)

        def start(c, buf):
            pltpu.async_copy(
                table_hbm.at[idx_v.at[pl.ds(c * CH, CH)]],
                rows_v.at[buf],
                sems[buf],
            )

        def wait(c, buf):
            pltpu.make_async_copy(
                table_hbm.at[idx_v.at[pl.ds(c * CH, CH)]],
                rows_v.at[buf],
                sems[buf],
            ).wait()

        start(0, 0)
        start(1, 1)

        def body(i, _):
            c0 = i * 2
            for b in range(2):
                c = c0 + b
                wait(c, b)
                pltpu.sync_copy(rows_v.at[b],
                                out_hbm.at[pl.ds(base + c * CH, CH)])

                @pl.when(c + 2 < NCHUNK)
                def _():
                    start(c + 2, b)
            return 0

        lax.fori_loop(0, NCHUNK // 2, body, 0)

    return gather_kernel


_gather = _make_gather()


_VB = 4096


def _prep_body(x_ref, o_ref):
    o_ref[...] = jnp.pad(x_ref[...].T, ((0, 0), (0, DP - D)))


_prep_tc = pl.pallas_call(
    _prep_body,
    grid=(pl.cdiv(VOCAB, _VB),),
    in_specs=[pl.BlockSpec((D, _VB), lambda g: (0, g))],
    out_specs=pl.BlockSpec((_VB, DP), lambda g: (g, 0)),
    out_shape=jax.ShapeDtypeStruct((VOCAB, DP), jnp.float32),
)


def _finish_body(x_ref, o_ref):
    o_ref[...] = x_ref[:, :D].T[None]


_finish_tc = pl.pallas_call(
    _finish_body,
    grid=(S,),
    in_specs=[pl.BlockSpec((B, DP), lambda g: (g, 0))],
    out_specs=pl.BlockSpec((1, D, B), lambda g: (g, 0, 0)),
    out_shape=jax.ShapeDtypeStruct((S, D, B), jnp.float32),
)


def kernel(indices, table):
    idx_flat = indices.T.reshape(-1).astype(jnp.int32)
    table_p = _prep_tc(table.T)
    y = _gather(idx_flat, table_p)
    tmp = _finish_tc(y)
    return tmp.transpose(2, 0, 1)
